# Initial kernel scaffold; baseline (speedup 1.0000x reference)
import jax
import jax.numpy as jnp
from jax.experimental import pallas as pl


def kernel(X, edge_index, batch, params):
    raise NotImplementedError("write your pallas kernel here")


# speedup vs baseline: 4.1105x; 4.1105x over previous
import functools

import jax
import jax.numpy as jnp
from jax import lax
from jax.experimental import pallas as pl
from jax.experimental.pallas import tpu as pltpu
from jax.experimental.pallas import tpu_sc as plsc

N = 10000
E = 320000
D = 128
G = 64
C = 10

NC = 2
NS = 16
NW = NC * NS
EPW = E // NW
CH = 80
NCHUNK = EPW // CH
RPS = 624
ZROWS = 104
TAIL = N - NS * RPS


_MM_ROWS = 2000


def _mm_body(h_ref, w_ref, b_ref, k_ref, q_ref, v_ref, s_ref):
    r = jnp.dot(h_ref[...], w_ref[...], preferred_element_type=jnp.float32)
    r = r + b_ref[...]
    k_ref[...] = r[:, 0 * D:1 * D]
    q_ref[...] = r[:, 1 * D:2 * D]
    v_ref[...] = r[:, 2 * D:3 * D]
    s_ref[...] = r[:, 3 * D:4 * D]


def _project(h, wcat, bcat):
    grid = N // _MM_ROWS
    out = jax.ShapeDtypeStruct((N, D), jnp.float32)
    return pl.pallas_call(
        _mm_body,
        grid=(grid,),
        in_specs=[
            pl.BlockSpec((_MM_ROWS, D), lambda i: (i, 0)),
            pl.BlockSpec((D, 4 * D), lambda i: (0, 0)),
            pl.BlockSpec((1, 4 * D), lambda i: (0, 0)),
        ],
        out_specs=[pl.BlockSpec((_MM_ROWS, D), lambda i: (i, 0))] * 4,
        out_shape=[out, out, out, out],
    )(h, wcat, bcat)


def _edge_body(k_hbm, q_hbm, v_hbm, src_hbm, dst_hbm, out_hbm,
               acc, zbuf, sidx, didx, kr, qr, vr, sem):
    c = lax.axis_index("c")
    s = lax.axis_index("s")
    wid = c * NS + s

    def _zrow(i, _):
        for j in range(D // 16):
            zbuf[i, pl.ds(j * 16, 16)] = jnp.zeros((16,), jnp.float32)
        return 0
    lax.fori_loop(0, ZROWS, _zrow, 0)
    for t in range(RPS // ZROWS):
        pltpu.sync_copy(zbuf, acc.at[pl.ds(s * RPS + t * ZROWS, ZROWS)])

    @pl.when(s == 0)
    def _():
        pltpu.sync_copy(zbuf.at[pl.ds(0, TAIL)], acc.at[pl.ds(NS * RPS, TAIL)])
    plsc.subcore_barrier()

    def _chunk(i, _):
        base = wid * EPW + i * CH
        pltpu.sync_copy(src_hbm.at[pl.ds(base, CH)], sidx)
        pltpu.sync_copy(dst_hbm.at[pl.ds(base, CH)], didx)
        pltpu.async_copy(k_hbm.at[didx], kr, sem).wait()
        pltpu.async_copy(q_hbm.at[sidx], qr, sem).wait()
        pltpu.async_copy(v_hbm.at[sidx], vr, sem).wait()

        def _edge(e, _):
            for j in range(D // 16):
                sl = pl.ds(j * 16, 16)
                g = 1.0 / (1.0 + jnp.exp(-(kr[e, sl] + qr[e, sl])))
                vr[e, sl] = g * vr[e, sl]
            return 0
        lax.fori_loop(0, CH, _edge, 0)
        pltpu.sync_copy(vr, acc.at[didx], add=True)
        return 0
    lax.fori_loop(0, NCHUNK, _chunk, 0)

    plsc.subcore_barrier()
    for t in range(RPS // ZROWS):
        pltpu.sync_copy(acc.at[pl.ds(s * RPS + t * ZROWS, ZROWS)],
                        out_hbm.at[c, pl.ds(s * RPS + t * ZROWS, ZROWS)])

    @pl.when(s == 0)
    def _():
        pltpu.sync_copy(acc.at[pl.ds(NS * RPS, TAIL)],
                        out_hbm.at[c, pl.ds(NS * RPS, TAIL)])


_edge_kernel = functools.partial(
    pl.kernel,
    out_type=jax.ShapeDtypeStruct((NC, N, D), jnp.float32),
    mesh=plsc.VectorSubcoreMesh(core_axis_name="c", subcore_axis_name="s"),
    scratch_types=[
        pltpu.VMEM_SHARED((N, D), jnp.float32),
        pltpu.VMEM((ZROWS, D), jnp.float32),
        pltpu.VMEM((CH,), jnp.int32),
        pltpu.VMEM((CH,), jnp.int32),
        pltpu.VMEM((CH, D), jnp.float32),
        pltpu.VMEM((CH, D), jnp.float32),
        pltpu.VMEM((CH, D), jnp.float32),
        pltpu.SemaphoreType.DMA,
    ],
)(_edge_body)


def _post_body(p_ref, s_ref, g_ref, b_ref, out_ref):
    x = p_ref[0] + p_ref[1] + s_ref[...]
    x = jnp.maximum(x, 0.0)
    mu = jnp.mean(x, axis=0, keepdims=True)
    var = jnp.mean(jnp.square(x - mu), axis=0, keepdims=True)
    out_ref[...] = (x - mu) * lax.rsqrt(var + 1e-5) * g_ref[...] + b_ref[...]


def _post(parts, skip, gamma, beta):
    return pl.pallas_call(
        _post_body,
        out_shape=jax.ShapeDtypeStruct((N, D), jnp.float32),
    )(parts, skip, gamma, beta)


def _head_body(h_ref, b_ref, w_ref, bias_ref, out_ref):
    h = h_ref[...]
    gids = lax.broadcasted_iota(jnp.int32, (G, N), 0)
    onehot = (b_ref[...] == gids).astype(jnp.float32)
    sums = jnp.dot(onehot, h, preferred_element_type=jnp.float32)
    counts = jnp.sum(onehot, axis=1, keepdims=True)
    pooled = sums / jnp.maximum(counts, 1.0)
    logits = jnp.dot(pooled, w_ref[...],
                     preferred_element_type=jnp.float32) + bias_ref[...]
    m = jnp.max(logits, axis=1, keepdims=True)
    e = jnp.exp(logits - m)
    out_ref[...] = e / jnp.sum(e, axis=1, keepdims=True)


def _head(h, batch2d, w, bias):
    return pl.pallas_call(
        _head_body,
        out_shape=jax.ShapeDtypeStruct((G, C), jnp.float32),
    )(h, batch2d, w, bias)


def kernel(X, edge_index, batch, params):
    src = edge_index[0]
    dst = edge_index[1]
    batch2d = batch.reshape(1, N).astype(jnp.int32)

    h = X
    for l in range(5):
        p = params["convs"][l]
        wcat = jnp.concatenate(
            [p["W_key"], p["W_query"], p["W_value"], p["W_skip"]], axis=1)
        bcat = jnp.concatenate(
            [p["b_key"], p["b_query"], p["b_value"], p["b_skip"]]).reshape(1, 4 * D)
        k, q, v, s = _project(h, wcat, bcat)
        parts = _edge_kernel(k, q, v, src, dst)
        bn = params["bns"][l]
        h = _post(parts, s, bn["gamma"].reshape(1, D), bn["beta"].reshape(1, D))

    return _head(h, batch2d, params["lin"]["W"],
                 params["lin"]["b"].reshape(1, C))


# speedup vs baseline: 8.8231x; 2.1465x over previous
import functools

import jax
import jax.numpy as jnp
from jax import lax
from jax.experimental import pallas as pl
from jax.experimental.pallas import tpu as pltpu
from jax.experimental.pallas import tpu_sc as plsc

N = 10000
E = 320000
D = 128
G = 64
C = 10

NC = 2
NS = 16
NW = NC * NS
EPW = E // NW
CH = 40
NCHUNK = EPW // CH
RPS = 624
ZROWS = 104
TAIL = N - NS * RPS
CPB = 25
NBLK = NCHUNK // CPB


_MM_ROWS = 2000


def _mm_body(h_ref, w_ref, b_ref, k_ref, q_ref, v_ref, s_ref):
    r = jnp.dot(h_ref[...], w_ref[...], preferred_element_type=jnp.float32)
    r = r + b_ref[...]
    k_ref[...] = r[:, 0 * D:1 * D]
    q_ref[...] = r[:, 1 * D:2 * D]
    v_ref[...] = r[:, 2 * D:3 * D]
    s_ref[...] = r[:, 3 * D:4 * D]


def _project(h, wcat, bcat):
    grid = N // _MM_ROWS
    out = jax.ShapeDtypeStruct((N, D), jnp.float32)
    return pl.pallas_call(
        _mm_body,
        grid=(grid,),
        in_specs=[
            pl.BlockSpec((_MM_ROWS, D), lambda i: (i, 0)),
            pl.BlockSpec((D, 4 * D), lambda i: (0, 0)),
            pl.BlockSpec((1, 4 * D), lambda i: (0, 0)),
        ],
        out_specs=[pl.BlockSpec((_MM_ROWS, D), lambda i: (i, 0))] * 4,
        out_shape=[out, out, out, out],
    )(h, wcat, bcat)


def _edge_body(k_hbm, q_hbm, v_hbm, src_hbm, dst_hbm, out_hbm,
               acc,
               si0, di0, si1, di1,
               kr0, qr0, vr0, sem0,
               kr1, qr1, vr1, sem1):
    c = lax.axis_index("c")
    s = lax.axis_index("s")
    wid = c * NS + s
    ibufs = ((si0, di0), (si1, di1))
    gbufs = ((kr0, qr0, vr0, sem0), (kr1, qr1, vr1, sem1))

    def _zrow(i, _):
        for j in range(D // 16):
            kr0[i, pl.ds(j * 16, 16)] = jnp.zeros((16,), jnp.float32)
        return 0
    lax.fori_loop(0, CH, _zrow, 0)
    for t in range(RPS // CH):
        pltpu.sync_copy(kr0, acc.at[pl.ds(s * RPS + t * CH, CH)])
    rem = RPS - (RPS // CH) * CH
    if rem:
        pltpu.sync_copy(kr0.at[pl.ds(0, rem)],
                        acc.at[pl.ds(s * RPS + (RPS // CH) * CH, rem)])

    @pl.when(s == 0)
    def _():
        pltpu.sync_copy(kr0.at[pl.ds(0, TAIL)], acc.at[pl.ds(NS * RPS, TAIL)])
    plsc.subcore_barrier()

    def _fire(si, di, ci, b):
        kr, qr, vr, sem = gbufs[b]
        pltpu.async_copy(k_hbm.at[di.at[ci]], kr, sem)
        pltpu.async_copy(q_hbm.at[si.at[ci]], qr, sem)
        pltpu.async_copy(v_hbm.at[si.at[ci]], vr, sem)

    def _consume(si, di, ci, b):
        kr, qr, vr, sem = gbufs[b]
        pltpu.make_async_copy(k_hbm.at[di.at[ci]], kr, sem).wait()
        pltpu.make_async_copy(q_hbm.at[si.at[ci]], qr, sem).wait()
        pltpu.make_async_copy(v_hbm.at[si.at[ci]], vr, sem).wait()

        def _edge(e, _):
            for j in range(D // 16):
                sl = pl.ds(j * 16, 16)
                g = 1.0 / (1.0 + jnp.exp(kr[e, sl] + qr[e, sl]))
                vr[e, sl] = g * vr[e, sl]
            return 0
        lax.fori_loop(0, CH, _edge, 0)
        pltpu.sync_copy(vr, acc.at[di.at[ci]], add=True)

    pltpu.sync_copy(src_hbm.at[wid, 0], si0)
    pltpu.sync_copy(dst_hbm.at[wid, 0], di0)

    def _block(blk, b2):
        si, di = ibufs[b2]
        sin, din = ibufs[1 - b2]

        @pl.when(blk + 1 < NBLK)
        def _():
            pltpu.sync_copy(src_hbm.at[wid, blk + 1], sin)
            pltpu.sync_copy(dst_hbm.at[wid, blk + 1], din)

        _fire(si, di, 0, 0)

        def _pair(j, _):
            _fire(si, di, 2 * j + 1, 1)
            _consume(si, di, 2 * j, 0)
            _fire(si, di, 2 * j + 2, 0)
            _consume(si, di, 2 * j + 1, 1)
            return 0
        lax.fori_loop(0, (CPB - 1) // 2, _pair, 0)
        _consume(si, di, CPB - 1, 0)

    def _two(t, _):
        _block(2 * t, 0)
        _block(2 * t + 1, 1)
        return 0
    lax.fori_loop(0, NBLK // 2, _two, 0)

    plsc.subcore_barrier()
    for t in range(RPS // ZROWS):
        pltpu.sync_copy(acc.at[pl.ds(s * RPS + t * ZROWS, ZROWS)],
                        out_hbm.at[c, pl.ds(s * RPS + t * ZROWS, ZROWS)])

    @pl.when(s == 0)
    def _():
        pltpu.sync_copy(acc.at[pl.ds(NS * RPS, TAIL)],
                        out_hbm.at[c, pl.ds(NS * RPS, TAIL)])


_edge_kernel = functools.partial(
    pl.kernel,
    out_type=jax.ShapeDtypeStruct((NC, N, D), jnp.float32),
    mesh=plsc.VectorSubcoreMesh(core_axis_name="c", subcore_axis_name="s"),
    scratch_types=[
        pltpu.VMEM_SHARED((N, D), jnp.float32),
        pltpu.VMEM((CPB, CH), jnp.int32),
        pltpu.VMEM((CPB, CH), jnp.int32),
        pltpu.VMEM((CPB, CH), jnp.int32),
        pltpu.VMEM((CPB, CH), jnp.int32),
    ] + 2 * [
        pltpu.VMEM((CH, D), jnp.float32),
        pltpu.VMEM((CH, D), jnp.float32),
        pltpu.VMEM((CH, D), jnp.float32),
        pltpu.SemaphoreType.DMA,
    ],
)(_edge_body)


def _post_body(p_ref, s_ref, g_ref, b_ref, out_ref):
    x = p_ref[0] + p_ref[1] + s_ref[...]
    x = jnp.maximum(x, 0.0)
    mu = jnp.mean(x, axis=0, keepdims=True)
    var = jnp.mean(jnp.square(x - mu), axis=0, keepdims=True)
    out_ref[...] = (x - mu) * lax.rsqrt(var + 1e-5) * g_ref[...] + b_ref[...]


def _post(parts, skip, gamma, beta):
    return pl.pallas_call(
        _post_body,
        out_shape=jax.ShapeDtypeStruct((N, D), jnp.float32),
    )(parts, skip, gamma, beta)


def _head_body(h_ref, b_ref, w_ref, bias_ref, out_ref):
    h = h_ref[...]
    gids = lax.broadcasted_iota(jnp.int32, (G, N), 0)
    onehot = (b_ref[...] == gids).astype(jnp.float32)
    sums = jnp.dot(onehot, h, preferred_element_type=jnp.float32)
    counts = jnp.sum(onehot, axis=1, keepdims=True)
    pooled = sums / jnp.maximum(counts, 1.0)
    logits = jnp.dot(pooled, w_ref[...],
                     preferred_element_type=jnp.float32) + bias_ref[...]
    m = jnp.max(logits, axis=1, keepdims=True)
    e = jnp.exp(logits - m)
    out_ref[...] = e / jnp.sum(e, axis=1, keepdims=True)


def _head(h, batch2d, w, bias):
    return pl.pallas_call(
        _head_body,
        out_shape=jax.ShapeDtypeStruct((G, C), jnp.float32),
    )(h, batch2d, w, bias)


def kernel(X, edge_index, batch, params):
    src = edge_index[0].reshape(NW, NBLK, CPB, CH)
    dst = edge_index[1].reshape(NW, NBLK, CPB, CH)
    batch2d = batch.reshape(1, N).astype(jnp.int32)

    h = X
    for l in range(5):
        p = params["convs"][l]
        wcat = jnp.concatenate(
            [-p["W_key"], -p["W_query"], p["W_value"], p["W_skip"]], axis=1)
        bcat = jnp.concatenate(
            [-p["b_key"], -p["b_query"], p["b_value"], p["b_skip"]]).reshape(1, 4 * D)
        k, q, v, s = _project(h, wcat, bcat)
        parts = _edge_kernel(k, q, v, src, dst)
        bn = params["bns"][l]
        h = _post(parts, s, bn["gamma"].reshape(1, D), bn["beta"].reshape(1, D))

    return _head(h, batch2d, params["lin"]["W"],
                 params["lin"]["b"].reshape(1, C))


# speedup vs baseline: 8.8805x; 1.0065x over previous
import functools

import jax
import jax.numpy as jnp
import numpy as np
from jax import lax
from jax.experimental import pallas as pl
from jax.experimental.pallas import tpu as pltpu
from jax.experimental.pallas import tpu_sc as plsc

N = 10000
E = 320000
D = 128
G = 64
C = 10

NC = 2
NS = 16
NW = NC * NS
EPW = E // NW
CH = 40
NCHUNK = EPW // CH
RPS = 624
ZROWS = 104
TAIL = N - NS * RPS
CPB = 25
NBLK = NCHUNK // CPB

_PLO = np.concatenate([32 * g + np.arange(16) for g in range(4)])
_PHI = _PLO + 16


_MM_ROWS = 2000


def _rnd_bf16_bits(x):
    u = lax.bitcast_convert_type(x, jnp.int32)
    return u + 0x7FFF + ((u >> 16) & 1)


def _mm_body(h_ref, w_ref, b_ref, kq_ref, v_ref, s_ref):
    r = jnp.dot(h_ref[...], w_ref[...], preferred_element_type=jnp.float32)
    r = r + b_ref[...]
    mask = jnp.int32(-65536)
    klo = _rnd_bf16_bits(r[:, 0:64])
    khi = _rnd_bf16_bits(r[:, 64:128])
    qlo = _rnd_bf16_bits(r[:, 128:192])
    qhi = _rnd_bf16_bits(r[:, 192:256])
    kw = (lax.shift_right_logical(klo, 16)) | (khi & mask)
    qw = (lax.shift_right_logical(qlo, 16)) | (qhi & mask)
    kq_ref[...] = jnp.concatenate([kw, qw], axis=1)
    v_ref[...] = r[:, 2 * D:3 * D]
    s_ref[...] = r[:, 3 * D:4 * D]


def _project(h, wcat, bcat):
    grid = N // _MM_ROWS
    return pl.pallas_call(
        _mm_body,
        grid=(grid,),
        in_specs=[
            pl.BlockSpec((_MM_ROWS, D), lambda i: (i, 0)),
            pl.BlockSpec((D, 4 * D), lambda i: (0, 0)),
            pl.BlockSpec((1, 4 * D), lambda i: (0, 0)),
        ],
        out_specs=[
            pl.BlockSpec((_MM_ROWS, D), lambda i: (i, 0)),
            pl.BlockSpec((_MM_ROWS, D), lambda i: (i, 0)),
            pl.BlockSpec((_MM_ROWS, D), lambda i: (i, 0)),
        ],
        out_shape=[
            jax.ShapeDtypeStruct((N, D), jnp.int32),
            jax.ShapeDtypeStruct((N, D), jnp.float32),
            jax.ShapeDtypeStruct((N, D), jnp.float32),
        ],
    )(h, wcat, bcat)


def _edge_body(kq_hbm, v_hbm, src_hbm, dst_hbm, out_hbm,
               acc,
               si0, di0, si1, di1,
               kd0, ks0, vr0, sem0,
               kd1, ks1, vr1, sem1):
    c = lax.axis_index("c")
    s = lax.axis_index("s")
    wid = c * NS + s
    ibufs = ((si0, di0), (si1, di1))
    gbufs = ((kd0, ks0, vr0, sem0), (kd1, ks1, vr1, sem1))

    def _zrow(i, _):
        for j in range(D // 16):
            vr0[i, pl.ds(j * 16, 16)] = jnp.zeros((16,), jnp.float32)
        return 0
    lax.fori_loop(0, CH, _zrow, 0)
    for t in range(RPS // CH):
        pltpu.sync_copy(vr0, acc.at[pl.ds(s * RPS + t * CH, CH)])
    rem = RPS - (RPS // CH) * CH
    if rem:
        pltpu.sync_copy(vr0.at[pl.ds(0, rem)],
                        acc.at[pl.ds(s * RPS + (RPS // CH) * CH, rem)])

    @pl.when(s == 0)
    def _():
        pltpu.sync_copy(vr0.at[pl.ds(0, TAIL)], acc.at[pl.ds(NS * RPS, TAIL)])
    plsc.subcore_barrier()

    def _fire(si, di, ci, b):
        kd, ks, vr, sem = gbufs[b]
        pltpu.async_copy(kq_hbm.at[di.at[ci]], kd, sem)
        pltpu.async_copy(kq_hbm.at[si.at[ci]], ks, sem)
        pltpu.async_copy(v_hbm.at[si.at[ci]], vr, sem)

    def _consume(si, di, ci, b):
        kd, ks, vr, sem = gbufs[b]
        pltpu.make_async_copy(kq_hbm.at[di.at[ci]], kd, sem).wait()
        pltpu.make_async_copy(kq_hbm.at[si.at[ci]], ks, sem).wait()
        pltpu.make_async_copy(v_hbm.at[si.at[ci]], vr, sem).wait()
        mask = jnp.int32(-65536)

        def _edge(e, _):
            for g in range(D // 32):
                wk = kd[e, pl.ds(g * 16, 16)]
                wq = ks[e, pl.ds(64 + g * 16, 16)]
                klo = lax.bitcast_convert_type(wk << 16, jnp.float32)
                khi = lax.bitcast_convert_type(wk & mask, jnp.float32)
                qlo = lax.bitcast_convert_type(wq << 16, jnp.float32)
                qhi = lax.bitcast_convert_type(wq & mask, jnp.float32)
                glo = 1.0 / (1.0 + jnp.exp(klo + qlo))
                ghi = 1.0 / (1.0 + jnp.exp(khi + qhi))
                lo = pl.ds(g * 32, 16)
                hi = pl.ds(g * 32 + 16, 16)
                vr[e, lo] = glo * vr[e, lo]
                vr[e, hi] = ghi * vr[e, hi]
            return 0
        lax.fori_loop(0, CH, _edge, 0)
        pltpu.sync_copy(vr, acc.at[di.at[ci]], add=True)

    pltpu.sync_copy(src_hbm.at[wid, 0], si0)
    pltpu.sync_copy(dst_hbm.at[wid, 0], di0)

    def _block(blk, b2):
        si, di = ibufs[b2]
        sin, din = ibufs[1 - b2]

        @pl.when(blk + 1 < NBLK)
        def _():
            pltpu.sync_copy(src_hbm.at[wid, blk + 1], sin)
            pltpu.sync_copy(dst_hbm.at[wid, blk + 1], din)

        _fire(si, di, 0, 0)

        def _pair(j, _):
            _fire(si, di, 2 * j + 1, 1)
            _consume(si, di, 2 * j, 0)
            _fire(si, di, 2 * j + 2, 0)
            _consume(si, di, 2 * j + 1, 1)
            return 0
        lax.fori_loop(0, (CPB - 1) // 2, _pair, 0)
        _consume(si, di, CPB - 1, 0)

    def _two(t, _):
        _block(2 * t, 0)
        _block(2 * t + 1, 1)
        return 0
    lax.fori_loop(0, NBLK // 2, _two, 0)

    plsc.subcore_barrier()
    for t in range(RPS // ZROWS):
        pltpu.sync_copy(acc.at[pl.ds(s * RPS + t * ZROWS, ZROWS)],
                        out_hbm.at[c, pl.ds(s * RPS + t * ZROWS, ZROWS)])

    @pl.when(s == 0)
    def _():
        pltpu.sync_copy(acc.at[pl.ds(NS * RPS, TAIL)],
                        out_hbm.at[c, pl.ds(NS * RPS, TAIL)])


_edge_kernel = functools.partial(
    pl.kernel,
    out_type=jax.ShapeDtypeStruct((NC, N, D), jnp.float32),
    mesh=plsc.VectorSubcoreMesh(core_axis_name="c", subcore_axis_name="s"),
    scratch_types=[
        pltpu.VMEM_SHARED((N, D), jnp.float32),
        pltpu.VMEM((CPB, CH), jnp.int32),
        pltpu.VMEM((CPB, CH), jnp.int32),
        pltpu.VMEM((CPB, CH), jnp.int32),
        pltpu.VMEM((CPB, CH), jnp.int32),
    ] + 2 * [
        pltpu.VMEM((CH, D), jnp.int32),
        pltpu.VMEM((CH, D), jnp.int32),
        pltpu.VMEM((CH, D), jnp.float32),
        pltpu.SemaphoreType.DMA,
    ],
)(_edge_body)


def _post_body(p_ref, s_ref, g_ref, b_ref, out_ref):
    x = p_ref[0] + p_ref[1] + s_ref[...]
    x = jnp.maximum(x, 0.0)
    mu = jnp.mean(x, axis=0, keepdims=True)
    var = jnp.mean(jnp.square(x - mu), axis=0, keepdims=True)
    out_ref[...] = (x - mu) * lax.rsqrt(var + 1e-5) * g_ref[...] + b_ref[...]


def _post(parts, skip, gamma, beta):
    return pl.pallas_call(
        _post_body,
        out_shape=jax.ShapeDtypeStruct((N, D), jnp.float32),
    )(parts, skip, gamma, beta)


def _head_body(h_ref, b_ref, w_ref, bias_ref, out_ref):
    h = h_ref[...]
    gids = lax.broadcasted_iota(jnp.int32, (G, N), 0)
    onehot = (b_ref[...] == gids).astype(jnp.float32)
    sums = jnp.dot(onehot, h, preferred_element_type=jnp.float32)
    counts = jnp.sum(onehot, axis=1, keepdims=True)
    pooled = sums / jnp.maximum(counts, 1.0)
    logits = jnp.dot(pooled, w_ref[...],
                     preferred_element_type=jnp.float32) + bias_ref[...]
    m = jnp.max(logits, axis=1, keepdims=True)
    e = jnp.exp(logits - m)
    out_ref[...] = e / jnp.sum(e, axis=1, keepdims=True)


def _head(h, batch2d, w, bias):
    return pl.pallas_call(
        _head_body,
        out_shape=jax.ShapeDtypeStruct((G, C), jnp.float32),
    )(h, batch2d, w, bias)


def kernel(X, edge_index, batch, params):
    src = edge_index[0].reshape(NW, NBLK, CPB, CH)
    dst = edge_index[1].reshape(NW, NBLK, CPB, CH)
    batch2d = batch.reshape(1, N).astype(jnp.int32)
    plo = jnp.asarray(_PLO)
    phi = jnp.asarray(_PHI)

    h = X
    for l in range(5):
        p = params["convs"][l]
        wcat = jnp.concatenate(
            [-p["W_key"][:, plo], -p["W_key"][:, phi],
             -p["W_query"][:, plo], -p["W_query"][:, phi],
             p["W_value"], p["W_skip"]], axis=1)
        bcat = jnp.concatenate(
            [-p["b_key"][plo], -p["b_key"][phi],
             -p["b_query"][plo], -p["b_query"][phi],
             p["b_value"], p["b_skip"]]).reshape(1, 4 * D)
        kq, v, sk = _project(h, wcat, bcat)
        parts = _edge_kernel(kq, v, src, dst)
        bn = params["bns"][l]
        h = _post(parts, sk, bn["gamma"].reshape(1, D), bn["beta"].reshape(1, D))

    return _head(h, batch2d, params["lin"]["W"],
                 params["lin"]["b"].reshape(1, C))
